# baseline (device time: 586649 ns/iter reference)
import jax
import jax.numpy as jnp
from jax import lax
from jax.experimental import pallas as pl
from jax.experimental.pallas import tpu as pltpu

N_DEV = 16
N_Z = 4
N_Q = 4
NSLOT = 2
S = 2

MESH = pl.DeviceIdType.MESH


def kernel(x, w_mat):
    m_per, k = x.shape
    _, n_per = w_mat.shape
    half = m_per // 2
    subm = half // S
    wchunk = k // 4

    def body(x_ref, w_ref, out_ref, colbuf, cf, cb, wb, xs, ws, obuf,
             us_send, us_recv, ds_send, ds_recv,
             sf_send, sf_recv, sb_send, sb_recv,
             cred_f, cred_b, stg_sem, osem):
        p = lax.axis_index("i")
        z = p // N_Q
        q = p % N_Q
        right = z * N_Q + (q + 1) % N_Q
        left = z * N_Q + (q + N_Q - 1) % N_Q
        up = p + N_Q
        down = p - N_Q
        has_up = z < N_Z - 1
        has_down = z > 0

        barrier = pltpu.get_barrier_semaphore()
        for nbr in (left, right):
            pl.semaphore_signal(barrier, inc=1, device_id=(nbr,),
                                device_id_type=MESH)

        @pl.when(has_up)
        def _():
            pl.semaphore_signal(barrier, inc=1, device_id=(up,),
                                device_id_type=MESH)

        @pl.when(has_down)
        def _():
            pl.semaphore_signal(barrier, inc=1, device_id=(down,),
                                device_id_type=MESH)

        n_z_nbrs = has_up.astype(jnp.int32) + has_down.astype(jnp.int32)

        @pl.when(n_z_nbrs == 1)
        def _():
            pl.semaphore_wait(barrier, 3)

        @pl.when(n_z_nbrs == 2)
        def _():
            pl.semaphore_wait(barrier, 4)

        def zrow(zidx, off, size):
            return pl.ds(zidx * m_per + off, size)

        def line_copy(src_z, sems, target):
            return pltpu.make_async_remote_copy(
                src_ref=colbuf.at[zrow(src_z, 0, m_per), :],
                dst_ref=colbuf.at[zrow(src_z, 0, m_per), :],
                send_sem=sems[0], recv_sem=sems[1],
                device_id=(target,), device_id_type=MESH)

        def zsrc_of(r):
            if r == 0:
                return z
            if r == 1:
                return jnp.where(z >= 1, z - 1, 1)
            if r == 2:
                return jnp.where(z <= 1, 2, jnp.where(z == 2, 3, 1))
            return jnp.where(z <= 1, 3, 0)

        p1_sends = []
        out_pending = [[], []]
        subm2 = subm // 2

        def stage_x(c):
            d = pltpu.make_async_copy(
                x_ref.at[c * subm2:(c + 1) * subm2, :], xs, stg_sem)
            d.start()
            d.wait()
            colbuf[zrow(z, c * subm2, subm2), :] = (
                xs[...].astype(jnp.bfloat16))

        def out_dma(par, j, obuf_off, out_row, size):
            d = pltpu.make_async_copy(
                obuf.at[par, pl.ds(obuf_off, size), :],
                out_ref.at[pl.ds(out_row, size), :],
                osem.at[par, j])
            d.start()
            out_pending[par].append(d)

        N_HOP = N_Q * (N_Q - 1)
        prev_f = [None] * S
        prev_b = [None] * S
        for H in range(N_HOP + 1):
            r, t = divmod(H, 3)
            cur = H % NSLOT
            nxt = (H + 1) % NSLOT
            par = H % 2
            send = H < N_HOP
            injection = (t == 0) and send

            if injection:
                zs = zsrc_of(r)
                if r > 0:
                    @pl.when(has_down & (z - r >= 0))
                    def _(r=r):
                        line_copy(z - r, (us_send.at[r - 1], us_recv.at[r - 1]),
                                  down).wait_recv()

                    @pl.when(has_up & (z + r <= N_Z - 1))
                    def _(r=r):
                        line_copy(z + r, (ds_send.at[r - 1], ds_recv.at[r - 1]),
                                  up).wait_recv()

                    if r <= 2:
                        up_cond = has_up & (z - r >= 0)
                        dn_cond = has_down & (z + r <= N_Z - 1)

                        @pl.when(up_cond)
                        def _(r=r):
                            line_copy(z - r, (us_send.at[r], us_recv.at[r]),
                                      up).start()

                        @pl.when(dn_cond)
                        def _(r=r):
                            line_copy(z + r, (ds_send.at[r], ds_recv.at[r]),
                                      down).start()

                        p1_sends.append(
                            (up_cond,
                             lambda r=r: line_copy(
                                 z - r, (us_send.at[r], us_recv.at[r]), up)))
                        p1_sends.append(
                            (dn_cond,
                             lambda r=r: line_copy(
                                 z + r, (ds_send.at[r], ds_recv.at[r]), down)))

            cur_f, cur_b = [None] * S, [None] * S
            for s in range(S):
                rows = slice(s * subm, (s + 1) * subm)
                if send:
                    if H >= 1:
                        pl.semaphore_wait(cred_f, 1)
                    if injection:
                        if H == 0:
                            stage_x(2 * s)
                            stage_x(2 * s + 1)
                        fsrc = colbuf.at[zrow(zs, s * subm, subm), :]
                    else:
                        prev_f[s].wait_recv()
                        fsrc = cf.at[cur, rows, :]
                    cur_f[s] = pltpu.make_async_remote_copy(
                        src_ref=fsrc, dst_ref=cf.at[nxt, rows, :],
                        send_sem=sf_send.at[cur, s], recv_sem=sf_recv.at[nxt, s],
                        device_id=(right,), device_id_type=MESH)
                    cur_f[s].start()
                    if H >= 1:
                        pl.semaphore_wait(cred_b, 1)
                    if injection:
                        if H == 0:
                            stage_x(2 * S + 2 * s)
                            stage_x(2 * S + 2 * s + 1)
                        bsrc = colbuf.at[zrow(zs, half + s * subm, subm), :]
                    else:
                        prev_b[s].wait_recv()
                        bsrc = cb.at[cur, rows, :]
                    cur_b[s] = pltpu.make_async_remote_copy(
                        src_ref=bsrc, dst_ref=cb.at[nxt, rows, :],
                        send_sem=sb_send.at[cur, s], recv_sem=sb_recv.at[nxt, s],
                        device_id=(left,), device_id_type=MESH)
                    cur_b[s].start()

            if H == 0:
                @pl.when(has_up)
                def _():
                    line_copy(z, (us_send.at[0], us_recv.at[0]), up).start()

                @pl.when(has_down)
                def _():
                    line_copy(z, (ds_send.at[0], ds_recv.at[0]), down).start()

                p1_sends.append(
                    (has_up,
                     lambda: line_copy(z, (us_send.at[0], us_recv.at[0]), up)))
                p1_sends.append(
                    (has_down,
                     lambda: line_copy(z, (ds_send.at[0], ds_recv.at[0]),
                                       down)))
                for wi in range(4):
                    d = pltpu.make_async_copy(
                        w_ref.at[wi * wchunk:(wi + 1) * wchunk, :], ws,
                        stg_sem)
                    d.start()
                    d.wait()
                    wb[wi * wchunk:(wi + 1) * wchunk, :] = (
                        ws[...].astype(jnp.bfloat16))

            if injection and H >= 1:
                for s in range(S):
                    prev_f[s].wait_recv()
                    prev_b[s].wait_recv()

            for d in out_pending[par]:
                d.wait()
            out_pending[par] = []

            if H >= 1:
                rp, tp = divmod(H - 1, 3)
                zsp = zsrc_of(rp)
                p_cw = N_Q * zsp + (q - tp - 1) % N_Q
                p_ccw = N_Q * zsp + (q + tp + 1) % N_Q

            for s in range(S):
                rows = slice(s * subm, (s + 1) * subm)
                if H >= 1:
                    if H == N_HOP:
                        prev_f[s].wait_recv()
                    obuf[par, pl.ds(s * subm, subm), :] = jnp.dot(
                        cf[cur, rows, :], wb[...],
                        preferred_element_type=jnp.float32)
                    if H == N_HOP:
                        prev_b[s].wait_recv()
                    obuf[par, pl.ds(half + s * subm, subm), :] = jnp.dot(
                        cb[cur, rows, :], wb[...],
                        preferred_element_type=jnp.float32)
                if send:
                    cur_f[s].wait_send()
                    cur_b[s].wait_send()
                    if H < N_HOP - 1:
                        pl.semaphore_signal(cred_f, inc=1, device_id=(left,),
                                            device_id_type=MESH)
                        pl.semaphore_signal(cred_b, inc=1, device_id=(right,),
                                            device_id_type=MESH)

            if H >= 1:
                out_dma(par, 0, 0, p_cw * m_per, half)
                out_dma(par, 1, half, p_ccw * m_per + half, half)

            if injection:
                p_inj = N_Q * zs + q
                obuf[par, pl.ds(m_per, m_per), :] = jnp.dot(
                    colbuf[zrow(zs, 0, m_per), :], wb[...],
                    preferred_element_type=jnp.float32)
                out_dma(par, 2, m_per, p_inj * m_per, m_per)

            if send:
                prev_f, prev_b = cur_f, cur_b

        for cond, mk in p1_sends:
            @pl.when(cond)
            def _(mk=mk):
                mk().wait_send()
        for par in (0, 1):
            for d in out_pending[par]:
                d.wait()

    return pl.pallas_call(
        body,
        out_shape=jax.ShapeDtypeStruct((N_DEV * m_per, n_per), jnp.float32),
        in_specs=[pl.BlockSpec(memory_space=pl.ANY),
                  pl.BlockSpec(memory_space=pl.ANY)],
        out_specs=pl.BlockSpec(memory_space=pl.ANY),
        scratch_shapes=[
            pltpu.VMEM((N_Z * m_per, k), jnp.bfloat16),
            pltpu.VMEM((NSLOT, half, k), jnp.bfloat16),
            pltpu.VMEM((NSLOT, half, k), jnp.bfloat16),
            pltpu.VMEM((k, n_per), jnp.bfloat16),
            pltpu.VMEM((subm // 2, k), jnp.float32),
            pltpu.VMEM((wchunk, n_per), jnp.float32),
            pltpu.VMEM((2, 2 * m_per, n_per), jnp.float32),
            pltpu.SemaphoreType.DMA((3,)),
            pltpu.SemaphoreType.DMA((3,)),
            pltpu.SemaphoreType.DMA((3,)),
            pltpu.SemaphoreType.DMA((3,)),
            pltpu.SemaphoreType.DMA((NSLOT, S)),
            pltpu.SemaphoreType.DMA((NSLOT, S)),
            pltpu.SemaphoreType.DMA((NSLOT, S)),
            pltpu.SemaphoreType.DMA((NSLOT, S)),
            pltpu.SemaphoreType.REGULAR,
            pltpu.SemaphoreType.REGULAR,
            pltpu.SemaphoreType.DMA,
            pltpu.SemaphoreType.DMA((2, 3)),
        ],
        compiler_params=pltpu.CompilerParams(
            collective_id=0, vmem_limit_bytes=62 * 1024 * 1024),
    )(x, w_mat)


# device time: 579233 ns/iter; 1.0128x vs baseline; 1.0128x over previous
import jax
import jax.numpy as jnp
from jax import lax
from jax.experimental import pallas as pl
from jax.experimental.pallas import tpu as pltpu

N_DEV = 16
N_Z = 4
N_Q = 4
NSLOT = 2
S = 2

MESH = pl.DeviceIdType.MESH


def kernel(x, w_mat):
    m_per, k = x.shape
    _, n_per = w_mat.shape
    half = m_per // 2
    subm = half // S
    wchunk = k // 4

    def body(x_ref, w_ref, out_ref, colbuf, cf, cb, wb, xs, ws, obuf,
             us_send, us_recv, ds_send, ds_recv,
             sf_send, sf_recv, sb_send, sb_recv,
             cred_f, cred_b, stg_sem, osem):
        p = lax.axis_index("i")
        z = p // N_Q
        q = p % N_Q
        right = z * N_Q + (q + 1) % N_Q
        left = z * N_Q + (q + N_Q - 1) % N_Q
        up = p + N_Q
        down = p - N_Q
        has_up = z < N_Z - 1
        has_down = z > 0

        barrier = pltpu.get_barrier_semaphore()
        for nbr in (left, right):
            pl.semaphore_signal(barrier, inc=1, device_id=(nbr,),
                                device_id_type=MESH)

        @pl.when(has_up)
        def _():
            pl.semaphore_signal(barrier, inc=1, device_id=(up,),
                                device_id_type=MESH)

        @pl.when(has_down)
        def _():
            pl.semaphore_signal(barrier, inc=1, device_id=(down,),
                                device_id_type=MESH)

        n_z_nbrs = has_up.astype(jnp.int32) + has_down.astype(jnp.int32)

        @pl.when(n_z_nbrs == 1)
        def _():
            pl.semaphore_wait(barrier, 3)

        @pl.when(n_z_nbrs == 2)
        def _():
            pl.semaphore_wait(barrier, 4)

        def zrow(zidx, off, size):
            return pl.ds(zidx * m_per + off, size)

        def line_copy(src_z, sems, target):
            return pltpu.make_async_remote_copy(
                src_ref=colbuf.at[zrow(src_z, 0, m_per), :],
                dst_ref=colbuf.at[zrow(src_z, 0, m_per), :],
                send_sem=sems[0], recv_sem=sems[1],
                device_id=(target,), device_id_type=MESH)

        def zsrc_of(r):
            if r == 0:
                return z
            if r == 1:
                return jnp.where(z >= 1, z - 1, 1)
            if r == 2:
                return jnp.where(z <= 1, 2, jnp.where(z == 2, 3, 1))
            return jnp.where(z <= 1, 3, 0)

        p1_sends = []
        out_pending = [[], []]

        def out_dma(par, j, obuf_off, out_row, size):
            d = pltpu.make_async_copy(
                obuf.at[par, pl.ds(obuf_off, size), :],
                out_ref.at[pl.ds(out_row, size), :],
                osem.at[par, j])
            d.start()
            out_pending[par].append(d)

        N_HOP = N_Q * (N_Q - 1)
        prev_f = [None] * S
        prev_b = [None] * S
        for H in range(N_HOP + 1):
            r, t = divmod(H, 3)
            cur = H % NSLOT
            nxt = (H + 1) % NSLOT
            par = H % 2
            send = H < N_HOP
            injection = (t == 0) and send

            if injection:
                zs = zsrc_of(r)
                if r == 0:
                    for ci in range(2 * S):
                        d = pltpu.make_async_copy(
                            x_ref.at[ci * subm:(ci + 1) * subm, :], xs,
                            stg_sem)
                        d.start()
                        d.wait()
                        colbuf[zrow(z, ci * subm, subm), :] = (
                            xs[...].astype(jnp.bfloat16))
                    @pl.when(has_up)
                    def _():
                        line_copy(z, (us_send.at[0], us_recv.at[0]),
                                  up).start()

                    @pl.when(has_down)
                    def _():
                        line_copy(z, (ds_send.at[0], ds_recv.at[0]),
                                  down).start()

                    p1_sends.append(
                        (has_up,
                         lambda: line_copy(z, (us_send.at[0], us_recv.at[0]),
                                           up)))
                    p1_sends.append(
                        (has_down,
                         lambda: line_copy(z, (ds_send.at[0], ds_recv.at[0]),
                                           down)))
                else:
                    @pl.when(has_down & (z - r >= 0))
                    def _(r=r):
                        line_copy(z - r, (us_send.at[r - 1], us_recv.at[r - 1]),
                                  down).wait_recv()

                    @pl.when(has_up & (z + r <= N_Z - 1))
                    def _(r=r):
                        line_copy(z + r, (ds_send.at[r - 1], ds_recv.at[r - 1]),
                                  up).wait_recv()

                    if r <= 2:
                        up_cond = has_up & (z - r >= 0)
                        dn_cond = has_down & (z + r <= N_Z - 1)

                        @pl.when(up_cond)
                        def _(r=r):
                            line_copy(z - r, (us_send.at[r], us_recv.at[r]),
                                      up).start()

                        @pl.when(dn_cond)
                        def _(r=r):
                            line_copy(z + r, (ds_send.at[r], ds_recv.at[r]),
                                      down).start()

                        p1_sends.append(
                            (up_cond,
                             lambda r=r: line_copy(
                                 z - r, (us_send.at[r], us_recv.at[r]), up)))
                        p1_sends.append(
                            (dn_cond,
                             lambda r=r: line_copy(
                                 z + r, (ds_send.at[r], ds_recv.at[r]), down)))

            if send and H >= 1:
                pl.semaphore_wait(cred_f, 1)
                pl.semaphore_wait(cred_b, 1)

            cur_f, cur_b = [None] * S, [None] * S
            for s in range(S):
                rows = slice(s * subm, (s + 1) * subm)
                if send:
                    if injection:
                        fsrc = colbuf.at[zrow(zs, s * subm, subm), :]
                    else:
                        prev_f[s].wait_recv()
                        fsrc = cf.at[cur, rows, :]
                    cur_f[s] = pltpu.make_async_remote_copy(
                        src_ref=fsrc, dst_ref=cf.at[nxt, rows, :],
                        send_sem=sf_send.at[cur, s], recv_sem=sf_recv.at[nxt, s],
                        device_id=(right,), device_id_type=MESH)
                    cur_f[s].start()
                    if injection:
                        bsrc = colbuf.at[zrow(zs, half + s * subm, subm), :]
                    else:
                        prev_b[s].wait_recv()
                        bsrc = cb.at[cur, rows, :]
                    cur_b[s] = pltpu.make_async_remote_copy(
                        src_ref=bsrc, dst_ref=cb.at[nxt, rows, :],
                        send_sem=sb_send.at[cur, s], recv_sem=sb_recv.at[nxt, s],
                        device_id=(left,), device_id_type=MESH)
                    cur_b[s].start()

            if H == 0:
                for wi in range(4):
                    d = pltpu.make_async_copy(
                        w_ref.at[wi * wchunk:(wi + 1) * wchunk, :], ws,
                        stg_sem)
                    d.start()
                    d.wait()
                    wb[wi * wchunk:(wi + 1) * wchunk, :] = (
                        ws[...].astype(jnp.bfloat16))

            if injection and H >= 1:
                for s in range(S):
                    prev_f[s].wait_recv()
                    prev_b[s].wait_recv()

            for d in out_pending[par]:
                d.wait()
            out_pending[par] = []

            if H >= 1:
                rp, tp = divmod(H - 1, 3)
                zsp = zsrc_of(rp)
                p_cw = N_Q * zsp + (q - tp - 1) % N_Q
                p_ccw = N_Q * zsp + (q + tp + 1) % N_Q
                if H == N_HOP:
                    for s in range(S):
                        rows = slice(s * subm, (s + 1) * subm)
                        prev_f[s].wait_recv()
                        obuf[par, pl.ds(s * subm, subm), :] = jnp.dot(
                            cf[cur, rows, :], wb[...],
                            preferred_element_type=jnp.float32)
                        prev_b[s].wait_recv()
                        obuf[par, pl.ds(half + s * subm, subm), :] = jnp.dot(
                            cb[cur, rows, :], wb[...],
                            preferred_element_type=jnp.float32)
                    out_dma(par, 0, 0, p_cw * m_per, half)
                    out_dma(par, 1, half, p_ccw * m_per + half, half)
                else:
                    obuf[par, :half, :] = jnp.dot(
                        cf[cur], wb[...], preferred_element_type=jnp.float32)
                    out_dma(par, 0, 0, p_cw * m_per, half)
                    obuf[par, pl.ds(half, half), :] = jnp.dot(
                        cb[cur], wb[...], preferred_element_type=jnp.float32)
                    out_dma(par, 1, half, p_ccw * m_per + half, half)

            if injection:
                p_inj = N_Q * zs + q
                obuf[par, pl.ds(m_per, m_per), :] = jnp.dot(
                    colbuf[zrow(zs, 0, m_per), :], wb[...],
                    preferred_element_type=jnp.float32)
                out_dma(par, 2, m_per, p_inj * m_per, m_per)

            if send:
                for s in range(S):
                    cur_f[s].wait_send()
                    cur_b[s].wait_send()
                if H < N_HOP - 1:
                    pl.semaphore_signal(cred_f, inc=1, device_id=(left,),
                                        device_id_type=MESH)
                    pl.semaphore_signal(cred_b, inc=1, device_id=(right,),
                                        device_id_type=MESH)
                prev_f, prev_b = cur_f, cur_b

        for cond, mk in p1_sends:
            @pl.when(cond)
            def _(mk=mk):
                mk().wait_send()
        for par in (0, 1):
            for d in out_pending[par]:
                d.wait()

    return pl.pallas_call(
        body,
        out_shape=jax.ShapeDtypeStruct((N_DEV * m_per, n_per), jnp.float32),
        in_specs=[pl.BlockSpec(memory_space=pl.ANY),
                  pl.BlockSpec(memory_space=pl.ANY)],
        out_specs=pl.BlockSpec(memory_space=pl.ANY),
        scratch_shapes=[
            pltpu.VMEM((N_Z * m_per, k), jnp.bfloat16),
            pltpu.VMEM((NSLOT, half, k), jnp.bfloat16),
            pltpu.VMEM((NSLOT, half, k), jnp.bfloat16),
            pltpu.VMEM((k, n_per), jnp.bfloat16),
            pltpu.VMEM((subm, k), jnp.float32),
            pltpu.VMEM((wchunk, n_per), jnp.float32),
            pltpu.VMEM((2, 2 * m_per, n_per), jnp.float32),
            pltpu.SemaphoreType.DMA((3,)),
            pltpu.SemaphoreType.DMA((3,)),
            pltpu.SemaphoreType.DMA((3,)),
            pltpu.SemaphoreType.DMA((3,)),
            pltpu.SemaphoreType.DMA((NSLOT, S)),
            pltpu.SemaphoreType.DMA((NSLOT, S)),
            pltpu.SemaphoreType.DMA((NSLOT, S)),
            pltpu.SemaphoreType.DMA((NSLOT, S)),
            pltpu.SemaphoreType.REGULAR,
            pltpu.SemaphoreType.REGULAR,
            pltpu.SemaphoreType.DMA,
            pltpu.SemaphoreType.DMA((2, 3)),
        ],
        compiler_params=pltpu.CompilerParams(
            collective_id=0, vmem_limit_bytes=62 * 1024 * 1024),
    )(x, w_mat)
